# baseline (speedup 1.0000x reference)
import functools

import jax
import jax.numpy as jnp
from jax import lax
from jax.experimental import pallas as pl
from jax.experimental.pallas import tpu as pltpu
from jax.experimental.pallas import tpu_sc as plsc

B = 1024
N_SPECIES = 1000
N_REAC = 20000
INV_DTG = 100.0

NC = 2
NSUB = 16
NW = NC * NSUB
ROWS = B // NW
LANES = 16
GRP = 8
CHUNK = 1024
LAST = N_REAC % CHUNK
NFULL = N_REAC // CHUNK
SPAD = 1024


def _sc_fused_body(rh_hbm, i0_hbm, i1_hbm, ea_hbm, lt_hbm, al_hbm, br_hbm,
                   t_hbm, g_hbm, out_hbm,
                   rh_v, t_v, g_v,
                   i0b, i1b, eab, ltb, alb, brb,
                   i0l, i1l, eal, ltl, all_, brl,
                   ob_v, ob_last_v, sem):
    wid = lax.axis_index("s") * NC + lax.axis_index("c")
    base = wid * ROWS

    pltpu.sync_copy(rh_hbm.at[pl.ds(base * SPAD, ROWS * SPAD)], rh_v)
    pltpu.sync_copy(t_hbm, t_v)
    pltpu.sync_copy(g_hbm, g_v)

    m_vecs = []
    s_vecs = []
    for h in range(2):
        t16 = t_v[pl.ds(base + h * LANES, LANES)]
        g16 = g_v[pl.ds(base + h * LANES, LANES)]
        m_vecs.append(-1.0 / t16)
        s_vecs.append(INV_DTG / g16)

    def run_chunk(c0, width, bufs, ob):
        i0c, i1c, eac, ltc, alc, brc = bufs
        descs = [
            pltpu.async_copy(i0_hbm.at[pl.ds(c0, width)], i0c, sem),
            pltpu.async_copy(i1_hbm.at[pl.ds(c0, width)], i1c, sem),
            pltpu.async_copy(ea_hbm.at[pl.ds(c0, width)], eac, sem),
            pltpu.async_copy(lt_hbm.at[pl.ds(c0, width)], ltc, sem),
            pltpu.async_copy(al_hbm.at[pl.ds(c0, width)], alc, sem),
            pltpu.async_copy(br_hbm.at[pl.ds(c0, width)], brc, sem),
        ]
        for d in descs:
            d.wait()

        for g in range(0, ROWS, GRP):
            m_sc = [m_vecs[(g + k) // LANES][(g + k) % LANES] for k in range(GRP)]
            s_sc = [s_vecs[(g + k) // LANES][(g + k) % LANES] for k in range(GRP)]

            def j_body(j, carry, g=g, m_sc=m_sc, s_sc=s_sc):
                o = j * LANES
                idx0 = i0c[pl.ds(o, LANES)]
                idx1 = i1c[pl.ds(o, LANES)]
                eaj = eac[pl.ds(o, LANES)]
                ltj = ltc[pl.ds(o, LANES)]
                cfj = alc[pl.ds(o, LANES)] * brc[pl.ds(o, LANES)]
                pairs = []
                for k in range(GRP):
                    row = rh_v.at[pl.ds((g + k) * SPAD, SPAD)]
                    pairs.append((plsc.load_gather(row, [idx0]),
                                  plsc.load_gather(row, [idx1])))
                for k in range(GRP):
                    bl = g + k
                    rh = pairs[k][0] + pairs[k][1]
                    lp = jnp.maximum(eaj * m_sc[k], ltj)
                    ob[bl, pl.ds(o, LANES)] = (cfj * s_sc[k]) * rh * jnp.exp(lp)
                return carry

            lax.fori_loop(0, width // LANES, j_body, 0)

        pltpu.sync_copy(ob, out_hbm.at[pl.ds(base, ROWS), pl.ds(c0, width)])

    def chunk_body(c, carry):
        run_chunk(c * CHUNK, CHUNK, (i0b, i1b, eab, ltb, alb, brb), ob_v)
        return carry

    lax.fori_loop(0, NFULL, chunk_body, 0)
    run_chunk(NFULL * CHUNK, LAST, (i0l, i1l, eal, ltl, all_, brl), ob_last_v)


@functools.cache
def _sc_fused_kernel():
    return pl.kernel(
        _sc_fused_body,
        out_type=jax.ShapeDtypeStruct((B, N_REAC), jnp.float32),
        mesh=plsc.VectorSubcoreMesh(
            core_axis_name="c", subcore_axis_name="s",
            num_cores=NC, num_subcores=NSUB,
        ),
        scratch_types=[
            pltpu.VMEM((ROWS * SPAD,), jnp.float32),
            pltpu.VMEM((B,), jnp.float32),
            pltpu.VMEM((B,), jnp.float32),
            pltpu.VMEM((CHUNK,), jnp.int32),
            pltpu.VMEM((CHUNK,), jnp.int32),
            pltpu.VMEM((CHUNK,), jnp.float32),
            pltpu.VMEM((CHUNK,), jnp.float32),
            pltpu.VMEM((CHUNK,), jnp.float32),
            pltpu.VMEM((CHUNK,), jnp.float32),
            pltpu.VMEM((LAST,), jnp.int32),
            pltpu.VMEM((LAST,), jnp.int32),
            pltpu.VMEM((LAST,), jnp.float32),
            pltpu.VMEM((LAST,), jnp.float32),
            pltpu.VMEM((LAST,), jnp.float32),
            pltpu.VMEM((LAST,), jnp.float32),
            pltpu.VMEM((ROWS, CHUNK), jnp.float32),
            pltpu.VMEM((ROWS, LAST), jnp.float32),
            pltpu.SemaphoreType.DMA,
        ],
        compiler_params=pltpu.CompilerParams(needs_layout_passes=False),
        name="sc_surface_reaction",
    )


@jax.jit
def kernel(rate_hopping, T_dust, den_gas, E_act, log_prob_surf_tunl, alpha,
           branching_ratio, inds_r):
    return _sc_fused_kernel()(
        jnp.pad(rate_hopping, ((0, 0), (0, SPAD - N_SPECIES))).reshape(B * SPAD),
        inds_r[:, 0], inds_r[:, 1],
        E_act, log_prob_surf_tunl, alpha, branching_ratio,
        T_dust.reshape(B), den_gas.reshape(B),
    )


# speedup vs baseline: 1.0396x; 1.0396x over previous
import functools

import jax
import jax.numpy as jnp
from jax import lax
from jax.experimental import pallas as pl
from jax.experimental.pallas import tpu as pltpu
from jax.experimental.pallas import tpu_sc as plsc

B = 1024
HALF = B // 2
N_SPECIES = 1000
N_REAC = 20000
INV_DTG = 100.0

NC = 2
NSUB = 16
NW = NC * NSUB
ROWS = HALF // NW
LANES = 16
GRP = 8
CHUNK = 1024
LAST = N_REAC % CHUNK
NFULL = N_REAC // CHUNK


def _sc_gather_body(rh_hbm, i0_hbm, i1_hbm, out_hbm, rh_v, i0_v, i1_v, ob_v,
                    ob_last_v):
    wid = lax.axis_index("s") * NC + lax.axis_index("c")
    base = wid * ROWS
    pltpu.sync_copy(rh_hbm.at[pl.ds(base * N_SPECIES, ROWS * N_SPECIES)], rh_v)
    pltpu.sync_copy(i0_hbm, i0_v)
    pltpu.sync_copy(i1_hbm, i1_v)

    def make_body(c0, buf):
        def j_body(j, carry):
            r0 = c0 + j * LANES
            idx0 = i0_v[pl.ds(r0, LANES)]
            idx1 = i1_v[pl.ds(r0, LANES)]
            for g in range(0, ROWS, GRP):
                pairs = []
                for bl in range(g, g + GRP):
                    v0 = plsc.load_gather(rh_v, [idx0 + bl * N_SPECIES])
                    v1 = plsc.load_gather(rh_v, [idx1 + bl * N_SPECIES])
                    pairs.append(v0 + v1)
                for bl, v in zip(range(g, g + GRP), pairs):
                    buf[bl, pl.ds(j * LANES, LANES)] = v
            return carry
        return j_body

    for c0 in range(0, NFULL * CHUNK, CHUNK):
        lax.fori_loop(0, CHUNK // LANES, make_body(c0, ob_v), 0)
        pltpu.sync_copy(ob_v, out_hbm.at[pl.ds(base, ROWS), pl.ds(c0, CHUNK)])

    c0 = N_REAC - LAST
    lax.fori_loop(0, LAST // LANES, make_body(c0, ob_last_v), 0)
    pltpu.sync_copy(ob_last_v, out_hbm.at[pl.ds(base, ROWS), pl.ds(c0, LAST)])


@functools.cache
def _sc_gather_kernel():
    return pl.kernel(
        _sc_gather_body,
        out_type=jax.ShapeDtypeStruct((HALF, N_REAC), jnp.float32),
        mesh=plsc.VectorSubcoreMesh(
            core_axis_name="c", subcore_axis_name="s",
            num_cores=NC, num_subcores=NSUB,
        ),
        scratch_types=[
            pltpu.VMEM((ROWS * N_SPECIES,), jnp.float32),
            pltpu.VMEM((N_REAC,), jnp.int32),
            pltpu.VMEM((N_REAC,), jnp.int32),
            pltpu.VMEM((ROWS, CHUNK), jnp.float32),
            pltpu.VMEM((ROWS, LAST), jnp.float32),
        ],
        compiler_params=pltpu.CompilerParams(needs_layout_passes=False),
        name="sc_gather_sum",
    )


BBLK = 64
NBLK_H = HALF // BBLK


def _tc_first_body(rh_ref, t_ref, g_ref, ea_ref, lt_ref, al_ref, br_ref, o_ref):
    inv_t = 1.0 / t_ref[...]
    scale = INV_DTG / g_ref[...]
    lp = jnp.maximum(-ea_ref[...] * inv_t, lt_ref[...])
    coef = al_ref[...] * br_ref[...]
    o_ref[...] = (coef * scale) * rh_ref[...] * jnp.exp(lp)


def _tc_second_body(rh_ref, t_ref, g_ref, ea_ref, lt_ref, al_ref, br_ref,
                    prev_ref, o_ref):
    _tc_first_body(rh_ref, t_ref, g_ref, ea_ref, lt_ref, al_ref, br_ref, o_ref)


_PSPEC = pl.BlockSpec((1, N_REAC), lambda i: (0, 0))
_HSPEC = [
    pl.BlockSpec((BBLK, N_REAC), lambda i: (i, 0)),
    pl.BlockSpec((BBLK, 1), lambda i: (i, 0)),
    pl.BlockSpec((BBLK, 1), lambda i: (i, 0)),
    _PSPEC, _PSPEC, _PSPEC, _PSPEC,
]


def _tc_first(rate_hop, t, g, ea, lt, al, br):
    return pl.pallas_call(
        _tc_first_body,
        grid=(NBLK_H,),
        in_specs=_HSPEC,
        out_specs=pl.BlockSpec((BBLK, N_REAC), lambda i: (i, 0)),
        out_shape=jax.ShapeDtypeStruct((B, N_REAC), jnp.float32),
    )(rate_hop, t, g, ea, lt, al, br)


def _tc_second(rate_hop, t, g, ea, lt, al, br, prev):
    return pl.pallas_call(
        _tc_second_body,
        grid=(NBLK_H,),
        in_specs=_HSPEC + [pl.BlockSpec((8, 128), lambda i: (0, 0))],
        out_specs=pl.BlockSpec((BBLK, N_REAC), lambda i: (i + NBLK_H, 0)),
        out_shape=jax.ShapeDtypeStruct((B, N_REAC), jnp.float32),
        input_output_aliases={7: 0},
    )(rate_hop, t, g, ea, lt, al, br, prev)


@jax.jit
def kernel(rate_hopping, T_dust, den_gas, E_act, log_prob_surf_tunl, alpha,
           branching_ratio, inds_r):
    i0 = inds_r[:, 0]
    i1 = inds_r[:, 1]
    rh_flat = rate_hopping.reshape(B * N_SPECIES)
    sc = _sc_gather_kernel()
    rh_a = sc(rh_flat[: HALF * N_SPECIES], i0, i1)
    rh_b = sc(rh_flat[HALF * N_SPECIES:], i0, i1)

    ea = E_act.reshape(1, N_REAC)
    lt = log_prob_surf_tunl.reshape(1, N_REAC)
    al = alpha.reshape(1, N_REAC)
    br = branching_ratio.reshape(1, N_REAC)
    out_a = _tc_first(rh_a, T_dust[:HALF], den_gas[:HALF], ea, lt, al, br)
    return _tc_second(rh_b, T_dust[HALF:], den_gas[HALF:], ea, lt, al, br,
                      out_a)


# speedup vs baseline: 1.1382x; 1.0948x over previous
import functools

import jax
import jax.numpy as jnp
from jax import lax
from jax.experimental import pallas as pl
from jax.experimental.pallas import tpu as pltpu
from jax.experimental.pallas import tpu_sc as plsc

B = 1024
N_SPECIES = 1000
N_REAC = 20000
INV_DTG = 100.0

NC = 2
NSUB = 16
NW = NC * NSUB
ROWS = B // NW
LANES = 16
CHUNK = 512
LAST = N_REAC % CHUNK
CHUNK_STARTS = tuple(range(0, N_REAC - LAST, CHUNK))


def _sc_gather_body(rh_hbm, i0_hbm, i1_hbm, out_hbm, rh_flat, i0_v, i1_v,
                    ob_a, ob_b, ob_last_v, sem_a, sem_b):
    wid = lax.axis_index("s") * NC + lax.axis_index("c")
    base = wid * ROWS
    pltpu.sync_copy(rh_hbm.at[pl.ds(base * N_SPECIES, ROWS * N_SPECIES)], rh_flat)
    pltpu.sync_copy(i0_hbm, i0_v)
    pltpu.sync_copy(i1_hbm, i1_v)

    GRP = 8

    def make_body(c0, buf):
        def j_body(j, carry):
            r0 = c0 + j * LANES
            idx0 = i0_v[pl.ds(r0, LANES)]
            idx1 = i1_v[pl.ds(r0, LANES)]
            for g in range(0, ROWS, GRP):
                pairs = []
                for bl in range(g, g + GRP):
                    v0 = plsc.load_gather(rh_flat, [idx0 + bl * N_SPECIES])
                    v1 = plsc.load_gather(rh_flat, [idx1 + bl * N_SPECIES])
                    pairs.append(v0 + v1)
                for bl, v in zip(range(g, g + GRP), pairs):
                    buf[bl, pl.ds(j * LANES, LANES)] = v
            return carry
        return j_body

    bufs = (ob_a, ob_b)
    sems = (sem_a, sem_b)
    descs = [None, None]
    for ci, c0 in enumerate(CHUNK_STARTS):
        p = ci % 2
        if descs[p] is not None:
            descs[p].wait()
        lax.fori_loop(0, CHUNK // LANES, make_body(c0, bufs[p]), 0)
        descs[p] = pltpu.async_copy(
            bufs[p], out_hbm.at[pl.ds(base, ROWS), pl.ds(c0, CHUNK)], sems[p]
        )

    c0 = N_REAC - LAST
    lax.fori_loop(0, LAST // LANES, make_body(c0, ob_last_v), 0)
    pltpu.sync_copy(ob_last_v, out_hbm.at[pl.ds(base, ROWS), pl.ds(c0, LAST)])
    for d in descs:
        d.wait()


@functools.cache
def _sc_gather_kernel():
    return pl.kernel(
        _sc_gather_body,
        out_type=jax.ShapeDtypeStruct((B, N_REAC), jnp.float32),
        mesh=plsc.VectorSubcoreMesh(
            core_axis_name="c", subcore_axis_name="s",
            num_cores=NC, num_subcores=NSUB,
        ),
        scratch_types=[
            pltpu.VMEM((ROWS * N_SPECIES,), jnp.float32),
            pltpu.VMEM((N_REAC,), jnp.int32),
            pltpu.VMEM((N_REAC,), jnp.int32),
            pltpu.VMEM((ROWS, CHUNK), jnp.float32),
            pltpu.VMEM((ROWS, CHUNK), jnp.float32),
            pltpu.VMEM((ROWS, LAST), jnp.float32),
            pltpu.SemaphoreType.DMA,
            pltpu.SemaphoreType.DMA,
        ],
        compiler_params=pltpu.CompilerParams(needs_layout_passes=False),
        name="sc_gather_sum",
    )


BBLK = 64


def _tc_elemwise_body(rh_ref, t_ref, g_ref, ea_ref, lt_ref, al_ref, br_ref, o_ref):
    inv_t = 1.0 / t_ref[...]
    scale = INV_DTG / g_ref[...]
    lp = jnp.maximum(-ea_ref[...] * inv_t, lt_ref[...])
    coef = al_ref[...] * br_ref[...]
    o_ref[...] = (coef * scale) * rh_ref[...] * jnp.exp(lp)


def _tc_elemwise(rate_hop, t_dust, den_gas, ea, lt, al, br):
    grid = (B // BBLK,)
    pspec = pl.BlockSpec((1, N_REAC), lambda i: (0, 0))
    return pl.pallas_call(
        _tc_elemwise_body,
        grid=grid,
        in_specs=[
            pl.BlockSpec((BBLK, N_REAC), lambda i: (i, 0)),
            pl.BlockSpec((BBLK, 1), lambda i: (i, 0)),
            pl.BlockSpec((BBLK, 1), lambda i: (i, 0)),
            pspec, pspec, pspec, pspec,
        ],
        out_specs=pl.BlockSpec((BBLK, N_REAC), lambda i: (i, 0)),
        out_shape=jax.ShapeDtypeStruct((B, N_REAC), jnp.float32),
    )(rate_hop, t_dust, den_gas, ea, lt, al, br)


@jax.jit
def kernel(rate_hopping, T_dust, den_gas, E_act, log_prob_surf_tunl, alpha,
           branching_ratio, inds_r):
    i0 = inds_r[:, 0]
    i1 = inds_r[:, 1]
    rate_hop = _sc_gather_kernel()(rate_hopping.reshape(B * N_SPECIES), i0, i1)
    return _tc_elemwise(
        rate_hop, T_dust, den_gas,
        E_act.reshape(1, N_REAC),
        log_prob_surf_tunl.reshape(1, N_REAC),
        alpha.reshape(1, N_REAC),
        branching_ratio.reshape(1, N_REAC),
    )


# speedup vs baseline: 1.1537x; 1.0137x over previous
import functools

import jax
import jax.numpy as jnp
from jax import lax
from jax.experimental import pallas as pl
from jax.experimental.pallas import tpu as pltpu
from jax.experimental.pallas import tpu_sc as plsc

B = 1024
N_SPECIES = 1000
N_REAC = 20000
INV_DTG = 100.0

NC = 2
NSUB = 16
NW = NC * NSUB
ROWS = B // NW
LANES = 16
CHUNK = 512
LAST = N_REAC % CHUNK
CHUNK_STARTS = tuple(range(0, N_REAC - LAST, CHUNK))


def _sc_gather_body(rh_hbm, i0_hbm, i1_hbm, out_hbm, rh_flat, i0_v, i1_v,
                    ob_a, ob_b, ob_last_v, sem_a, sem_b):
    wid = lax.axis_index("s") * NC + lax.axis_index("c")
    base = wid * ROWS
    pltpu.sync_copy(rh_hbm.at[pl.ds(base * N_SPECIES, ROWS * N_SPECIES)], rh_flat)
    pltpu.sync_copy(i0_hbm, i0_v)
    pltpu.sync_copy(i1_hbm, i1_v)

    GRP = 8

    def make_body(c0, buf):
        def j_body(j, carry):
            r0 = c0 + j * LANES
            idx0 = i0_v[pl.ds(r0, LANES)]
            idx1 = i1_v[pl.ds(r0, LANES)]
            for g in range(0, ROWS, GRP):
                pairs = []
                for bl in range(g, g + GRP):
                    v0 = plsc.load_gather(rh_flat, [idx0 + bl * N_SPECIES])
                    v1 = plsc.load_gather(rh_flat, [idx1 + bl * N_SPECIES])
                    pairs.append(v0 + v1)
                for bl, v in zip(range(g, g + GRP), pairs):
                    buf[bl, pl.ds(j * LANES, LANES)] = v
            return carry
        return j_body

    bufs = (ob_a, ob_b)
    sems = (sem_a, sem_b)
    descs = [None, None]
    for ci, c0 in enumerate(CHUNK_STARTS):
        p = ci % 2
        if descs[p] is not None:
            descs[p].wait()
        lax.fori_loop(0, CHUNK // LANES, make_body(c0, bufs[p]), 0)
        descs[p] = pltpu.async_copy(
            bufs[p], out_hbm.at[pl.ds(base, ROWS), pl.ds(c0, CHUNK)], sems[p]
        )

    c0 = N_REAC - LAST
    lax.fori_loop(0, LAST // LANES, make_body(c0, ob_last_v), 0)
    pltpu.sync_copy(ob_last_v, out_hbm.at[pl.ds(base, ROWS), pl.ds(c0, LAST)])
    for d in descs:
        d.wait()


@functools.cache
def _sc_gather_kernel():
    return pl.kernel(
        _sc_gather_body,
        out_type=jax.ShapeDtypeStruct((B, N_REAC), jnp.float32),
        mesh=plsc.VectorSubcoreMesh(
            core_axis_name="c", subcore_axis_name="s",
            num_cores=NC, num_subcores=NSUB,
        ),
        scratch_types=[
            pltpu.VMEM((ROWS * N_SPECIES,), jnp.float32),
            pltpu.VMEM((N_REAC,), jnp.int32),
            pltpu.VMEM((N_REAC,), jnp.int32),
            pltpu.VMEM((ROWS, CHUNK), jnp.float32),
            pltpu.VMEM((ROWS, CHUNK), jnp.float32),
            pltpu.VMEM((ROWS, LAST), jnp.float32),
            pltpu.SemaphoreType.DMA,
            pltpu.SemaphoreType.DMA,
        ],
        compiler_params=pltpu.CompilerParams(needs_layout_passes=False),
        name="sc_gather_sum",
    )


BBLK = 128


def _tc_elemwise_body(rh_ref, t_ref, g_ref, ea_ref, lt_ref, al_ref, br_ref, o_ref):
    inv_t = 1.0 / t_ref[...]
    scale = INV_DTG / g_ref[...]
    lp = jnp.maximum(-ea_ref[...] * inv_t, lt_ref[...])
    coef = al_ref[...] * br_ref[...]
    o_ref[...] = (coef * scale) * rh_ref[...] * jnp.exp(lp)


def _tc_elemwise(rate_hop, t_dust, den_gas, ea, lt, al, br):
    grid = (B // BBLK,)
    pspec = pl.BlockSpec((1, N_REAC), lambda i: (0, 0))
    return pl.pallas_call(
        _tc_elemwise_body,
        grid=grid,
        in_specs=[
            pl.BlockSpec((BBLK, N_REAC), lambda i: (i, 0)),
            pl.BlockSpec((BBLK, 1), lambda i: (i, 0)),
            pl.BlockSpec((BBLK, 1), lambda i: (i, 0)),
            pspec, pspec, pspec, pspec,
        ],
        out_specs=pl.BlockSpec((BBLK, N_REAC), lambda i: (i, 0)),
        out_shape=jax.ShapeDtypeStruct((B, N_REAC), jnp.float32),
        compiler_params=pltpu.CompilerParams(vmem_limit_bytes=100 * 1024 * 1024),
    )(rate_hop, t_dust, den_gas, ea, lt, al, br)


@jax.jit
def kernel(rate_hopping, T_dust, den_gas, E_act, log_prob_surf_tunl, alpha,
           branching_ratio, inds_r):
    i0 = inds_r[:, 0]
    i1 = inds_r[:, 1]
    rate_hop = _sc_gather_kernel()(rate_hopping.reshape(B * N_SPECIES), i0, i1)
    return _tc_elemwise(
        rate_hop, T_dust, den_gas,
        E_act.reshape(1, N_REAC),
        log_prob_surf_tunl.reshape(1, N_REAC),
        alpha.reshape(1, N_REAC),
        branching_ratio.reshape(1, N_REAC),
    )


# speedup vs baseline: 1.1691x; 1.0133x over previous
import functools

import jax
import jax.numpy as jnp
from jax import lax
from jax.experimental import pallas as pl
from jax.experimental.pallas import tpu as pltpu
from jax.experimental.pallas import tpu_sc as plsc

B = 1024
N_SPECIES = 1000
N_REAC = 20000
INV_DTG = 100.0

NC = 2
NSUB = 16
NW = NC * NSUB
ROWS = B // NW
LANES = 16
CHUNK = 512
LAST = N_REAC % CHUNK
CHUNK_STARTS = tuple(range(0, N_REAC - LAST, CHUNK))


def _sc_gather_body(rh_hbm, i0_hbm, i1_hbm, out_hbm, rh_flat, i0_v, i1_v,
                    ob_a, ob_b, ob_last_v, sem_a, sem_b):
    wid = lax.axis_index("s") * NC + lax.axis_index("c")
    base = wid * ROWS
    pltpu.sync_copy(rh_hbm.at[pl.ds(base * N_SPECIES, ROWS * N_SPECIES)], rh_flat)
    pltpu.sync_copy(i0_hbm, i0_v)
    pltpu.sync_copy(i1_hbm, i1_v)

    GRP = 16

    def make_body(c0, buf):
        def j_body(j, carry):
            r0 = c0 + j * LANES
            idx0 = i0_v[pl.ds(r0, LANES)]
            idx1 = i1_v[pl.ds(r0, LANES)]
            for g in range(0, ROWS, GRP):
                pairs = []
                for bl in range(g, g + GRP):
                    v0 = plsc.load_gather(rh_flat, [idx0 + bl * N_SPECIES])
                    v1 = plsc.load_gather(rh_flat, [idx1 + bl * N_SPECIES])
                    pairs.append(v0 + v1)
                for bl, v in zip(range(g, g + GRP), pairs):
                    buf[bl, pl.ds(j * LANES, LANES)] = v
            return carry
        return j_body

    bufs = (ob_a, ob_b)
    sems = (sem_a, sem_b)
    descs = [None, None]
    for ci, c0 in enumerate(CHUNK_STARTS):
        p = ci % 2
        if descs[p] is not None:
            descs[p].wait()
        lax.fori_loop(0, CHUNK // LANES, make_body(c0, bufs[p]), 0)
        descs[p] = pltpu.async_copy(
            bufs[p], out_hbm.at[pl.ds(base, ROWS), pl.ds(c0, CHUNK)], sems[p]
        )

    c0 = N_REAC - LAST
    lax.fori_loop(0, LAST // LANES, make_body(c0, ob_last_v), 0)
    pltpu.sync_copy(ob_last_v, out_hbm.at[pl.ds(base, ROWS), pl.ds(c0, LAST)])
    for d in descs:
        d.wait()


@functools.cache
def _sc_gather_kernel():
    return pl.kernel(
        _sc_gather_body,
        out_type=jax.ShapeDtypeStruct((B, N_REAC), jnp.float32),
        mesh=plsc.VectorSubcoreMesh(
            core_axis_name="c", subcore_axis_name="s",
            num_cores=NC, num_subcores=NSUB,
        ),
        scratch_types=[
            pltpu.VMEM((ROWS * N_SPECIES,), jnp.float32),
            pltpu.VMEM((N_REAC,), jnp.int32),
            pltpu.VMEM((N_REAC,), jnp.int32),
            pltpu.VMEM((ROWS, CHUNK), jnp.float32),
            pltpu.VMEM((ROWS, CHUNK), jnp.float32),
            pltpu.VMEM((ROWS, LAST), jnp.float32),
            pltpu.SemaphoreType.DMA,
            pltpu.SemaphoreType.DMA,
        ],
        compiler_params=pltpu.CompilerParams(needs_layout_passes=False),
        name="sc_gather_sum",
    )


BBLK = 128


def _tc_elemwise_body(rh_ref, t_ref, g_ref, ea_ref, lt_ref, al_ref, br_ref, o_ref):
    inv_t = 1.0 / t_ref[...]
    scale = INV_DTG / g_ref[...]
    lp = jnp.maximum(-ea_ref[...] * inv_t, lt_ref[...])
    coef = al_ref[...] * br_ref[...]
    o_ref[...] = (coef * scale) * rh_ref[...] * jnp.exp(lp)


def _tc_elemwise(rate_hop, t_dust, den_gas, ea, lt, al, br):
    grid = (B // BBLK,)
    pspec = pl.BlockSpec((1, N_REAC), lambda i: (0, 0))
    return pl.pallas_call(
        _tc_elemwise_body,
        grid=grid,
        in_specs=[
            pl.BlockSpec((BBLK, N_REAC), lambda i: (i, 0)),
            pl.BlockSpec((BBLK, 1), lambda i: (i, 0)),
            pl.BlockSpec((BBLK, 1), lambda i: (i, 0)),
            pspec, pspec, pspec, pspec,
        ],
        out_specs=pl.BlockSpec((BBLK, N_REAC), lambda i: (i, 0)),
        out_shape=jax.ShapeDtypeStruct((B, N_REAC), jnp.float32),
        compiler_params=pltpu.CompilerParams(vmem_limit_bytes=100 * 1024 * 1024),
    )(rate_hop, t_dust, den_gas, ea, lt, al, br)


@jax.jit
def kernel(rate_hopping, T_dust, den_gas, E_act, log_prob_surf_tunl, alpha,
           branching_ratio, inds_r):
    i0 = inds_r[:, 0]
    i1 = inds_r[:, 1]
    rate_hop = _sc_gather_kernel()(rate_hopping.reshape(B * N_SPECIES), i0, i1)
    return _tc_elemwise(
        rate_hop, T_dust, den_gas,
        E_act.reshape(1, N_REAC),
        log_prob_surf_tunl.reshape(1, N_REAC),
        alpha.reshape(1, N_REAC),
        branching_ratio.reshape(1, N_REAC),
    )
